# baseline (speedup 1.0000x reference)
import functools

import numpy as np
import jax
import jax.numpy as jnp
from jax import lax
from jax.experimental import pallas as pl
from jax.experimental.pallas import tpu as pltpu
from jax.experimental.pallas import tpu_sc as plsc

LATENT = 64
DRAWS = 100
BATCH = 16384
BLK = 128

_KS0 = np.uint32(0)
_KS1 = np.uint32(42)
_KS2 = np.uint32(0x1BD11BDA ^ 42)
_TINY = np.float32(np.finfo(np.float32).tiny)
_ROTS = ((13, 15, 26, 6), (17, 29, 16, 24))


def _rotl(x, r):
    return jnp.left_shift(x, np.uint32(r)) | jnp.right_shift(x, np.uint32(32 - r))


def _threefry_bits(n):
    ks = (_KS0, _KS1, _KS2)
    x0 = jnp.zeros_like(n)
    x1 = n + _KS1
    for i in range(5):
        for r in _ROTS[i % 2]:
            x0 = x0 + x1
            x1 = _rotl(x1, r)
            x1 = x1 ^ x0
        x0 = x0 + ks[(i + 1) % 3]
        x1 = x1 + ks[(i + 2) % 3] + np.uint32(i + 1)
    return x0 ^ x1


def _uniform(bits):
    fb = jnp.right_shift(bits, np.uint32(9)) | np.uint32(0x3F800000)
    f = lax.bitcast_convert_type(fb, jnp.float32) - np.float32(1.0)
    return jnp.maximum(_TINY, f + _TINY)


def _sample_body(batch, blk, embs_ref, out_ref):
    lt = embs_ref[...].T
    m = jnp.max(lt, axis=0, keepdims=True)
    t = jnp.exp(lt - m)
    s = jnp.sum(t, axis=0, keepdims=True)
    probs = t / s
    rinv = np.float32(1.0) / t

    col = lax.broadcasted_iota(jnp.int32, (LATENT, blk), 1) + pl.program_id(0) * blk
    row = lax.broadcasted_iota(jnp.int32, (LATENT, blk), 0)
    base = (col * LATENT + row).astype(jnp.uint32)
    jio = row

    def body(d, counts):
        n = base + (d * np.int32(batch * LATENT)).astype(jnp.uint32)
        v = -jnp.log(_uniform(_threefry_bits(n))) * rinv
        key = (lax.bitcast_convert_type(v, jnp.int32) & np.int32(~63)) | jio
        kmin = jnp.min(key, axis=0, keepdims=True)
        return counts + (key == kmin).astype(jnp.float32)

    counts = lax.fori_loop(0, DRAWS, body, jnp.zeros((LATENT, blk), jnp.float32))
    out_ref[...] = (counts * probs * np.float32(1.0 / DRAWS)).T


def _make_sampler(batch, blk, interpret=False):
    return pl.pallas_call(
        functools.partial(_sample_body, batch, blk),
        grid=(batch // blk,),
        in_specs=[pl.BlockSpec((blk, LATENT), lambda i: (i, 0))],
        out_specs=pl.BlockSpec((blk, LATENT), lambda i: (i, 0)),
        out_shape=jax.ShapeDtypeStruct((batch, LATENT), jnp.float32),
        interpret=interpret,
    )


_NW = 32
_BPW = BATCH // _NW
_CHUNK = 128
_NCHUNK = _BPW // _CHUNK


def _gather_body(table_hbm, idx_hbm, out_hbm, idx_v, rows_v, sem):
    wid = lax.axis_index("s") * 2 + lax.axis_index("c")
    pltpu.sync_copy(idx_hbm.at[pl.ds(wid * _NCHUNK, _NCHUNK)], idx_v)
    cps = [
        pltpu.async_copy(
            table_hbm.at[idx_v.at[j]], rows_v.at[pl.ds(j * _CHUNK, _CHUNK)], sem
        )
        for j in range(_NCHUNK)
    ]
    for c in cps:
        c.wait()
    pltpu.sync_copy(rows_v, out_hbm.at[pl.ds(wid * _BPW, _BPW)])


def _sc_gather(table, idx2d):
    mesh = plsc.VectorSubcoreMesh(core_axis_name="c", subcore_axis_name="s")
    return pl.kernel(
        _gather_body,
        mesh=mesh,
        out_type=jax.ShapeDtypeStruct((BATCH, LATENT), jnp.float32),
        scratch_types=[
            pltpu.VMEM((_NCHUNK, _CHUNK), jnp.int32),
            pltpu.VMEM((_BPW, LATENT), jnp.float32),
            pltpu.SemaphoreType.DMA,
        ],
    )(table, idx2d)


def kernel(labels, table):
    idx = labels.astype(jnp.int32)
    embs = jnp.take(table, idx, axis=0)
    return _make_sampler(BATCH, BLK)(embs)


# speedup vs baseline: 1.0797x; 1.0797x over previous
import functools

import numpy as np
import jax
import jax.numpy as jnp
from jax import lax
from jax.experimental import pallas as pl
from jax.experimental.pallas import tpu as pltpu
from jax.experimental.pallas import tpu_sc as plsc

LATENT = 64
DRAWS = 100
BATCH = 16384
BLK = 256

_KS0 = np.uint32(0)
_KS1 = np.uint32(42)
_KS2 = np.uint32(0x1BD11BDA ^ 42)
_TINY = np.float32(np.finfo(np.float32).tiny)
_ROTS = ((13, 15, 26, 6), (17, 29, 16, 24))


def _rotl(x, r):
    return jnp.left_shift(x, np.uint32(r)) | jnp.right_shift(x, np.uint32(32 - r))


def _threefry_bits(n):
    ks = (_KS0, _KS1, _KS2)
    x0 = jnp.zeros_like(n)
    x1 = n + _KS1
    for i in range(5):
        for r in _ROTS[i % 2]:
            x0 = x0 + x1
            x1 = _rotl(x1, r)
            x1 = x1 ^ x0
        x0 = x0 + ks[(i + 1) % 3]
        x1 = x1 + ks[(i + 2) % 3] + np.uint32(i + 1)
    return x0 ^ x1


def _uniform(bits):
    fb = jnp.right_shift(bits, np.uint32(9)) | np.uint32(0x3F800000)
    f = lax.bitcast_convert_type(fb, jnp.float32) - np.float32(1.0)
    return jnp.maximum(f, _TINY)


def _sample_body(batch, blk, embs_ref, out_ref):
    lt = embs_ref[...].T
    m = jnp.max(lt, axis=0, keepdims=True)
    t = jnp.exp(lt - m)
    s = jnp.sum(t, axis=0, keepdims=True)
    probs = t / s
    nrinv = np.float32(-1.0) / t

    col = lax.broadcasted_iota(jnp.int32, (LATENT, blk), 1) + pl.program_id(0) * blk
    row = lax.broadcasted_iota(jnp.int32, (LATENT, blk), 0)
    base = (col * LATENT + row).astype(jnp.uint32)
    jio = row
    stride = np.int32(batch * LATENT)

    def one_draw(n, counts):
        v = jnp.log(_uniform(_threefry_bits(n))) * nrinv
        key = (lax.bitcast_convert_type(v, jnp.int32) & np.int32(~63)) | jio
        kmin = jnp.min(key, axis=0, keepdims=True)
        return counts + (key == kmin).astype(jnp.float32)

    def body(k, counts):
        n = base + ((2 * k) * stride).astype(jnp.uint32)
        counts = one_draw(n, counts)
        return one_draw(n + stride.astype(jnp.uint32), counts)

    counts = lax.fori_loop(0, DRAWS // 2, body, jnp.zeros((LATENT, blk), jnp.float32))
    out_ref[...] = (counts * probs * np.float32(1.0 / DRAWS)).T


def _make_sampler(batch, blk, interpret=False):
    return pl.pallas_call(
        functools.partial(_sample_body, batch, blk),
        grid=(batch // blk,),
        in_specs=[pl.BlockSpec((blk, LATENT), lambda i: (i, 0))],
        out_specs=pl.BlockSpec((blk, LATENT), lambda i: (i, 0)),
        out_shape=jax.ShapeDtypeStruct((batch, LATENT), jnp.float32),
        interpret=interpret,
    )


_NW = 32
_BPW = BATCH // _NW
_CHUNK = 128
_NCHUNK = _BPW // _CHUNK


def _gather_body(table_hbm, idx_hbm, out_hbm, idx_v, rows_v, sem):
    wid = lax.axis_index("s") * 2 + lax.axis_index("c")
    pltpu.sync_copy(idx_hbm.at[pl.ds(wid * _NCHUNK, _NCHUNK)], idx_v)
    cps = [
        pltpu.async_copy(
            table_hbm.at[idx_v.at[j]], rows_v.at[pl.ds(j * _CHUNK, _CHUNK)], sem
        )
        for j in range(_NCHUNK)
    ]
    for c in cps:
        c.wait()
    pltpu.sync_copy(rows_v, out_hbm.at[pl.ds(wid * _BPW, _BPW)])


def _sc_gather(table, idx2d):
    mesh = plsc.VectorSubcoreMesh(core_axis_name="c", subcore_axis_name="s")
    return pl.kernel(
        _gather_body,
        mesh=mesh,
        out_type=jax.ShapeDtypeStruct((BATCH, LATENT), jnp.float32),
        scratch_types=[
            pltpu.VMEM((_NCHUNK, _CHUNK), jnp.int32),
            pltpu.VMEM((_BPW, LATENT), jnp.float32),
            pltpu.SemaphoreType.DMA,
        ],
    )(table, idx2d)


def kernel(labels, table):
    idx = labels.astype(jnp.int32)
    embs = jnp.take(table, idx, axis=0)
    return _make_sampler(BATCH, BLK)(embs)
